# baseline (speedup 1.0000x reference)
import functools

import jax
import jax.numpy as jnp
from jax import lax
from jax.experimental import pallas as pl
from jax.experimental.pallas import tpu as pltpu
from jax.experimental.pallas import tpu_sc as plsc

T = 2048
D = 1024
F = 512
E = 63
EP = 64
A = 2 * T
NTOT = A + T
BLK = 384
NB = NTOT // BLK
G = NB + EP

_NC = 2
_NS = 16
_NW = _NC * _NS


def _fiota(shape, dim):
    return lax.broadcasted_iota(jnp.int32, shape, dim).astype(jnp.float32)


def _router_body(p_ref, pos_ref, w_ref, b_ref, e_ref, f_ref, s_ref,
                 en_ref):
    probs = p_ref[...]
    col = lax.broadcasted_iota(jnp.int32, (T, EP), 1)

    big = jnp.int32(1 << 20)
    m1 = jnp.max(probs, axis=1, keepdims=True)
    i1 = jnp.min(jnp.where(probs == m1, col, big), axis=1, keepdims=True)
    pm = jnp.where(col == i1, -1.0, probs)
    m2 = jnp.max(pm, axis=1, keepdims=True)
    i2 = jnp.min(jnp.where(pm == m2, col, big), axis=1, keepdims=True)
    ssum = m1 + m2

    oh1 = (col == i1).astype(jnp.bfloat16)
    oh2 = (col == i2).astype(jnp.bfloat16)
    r = lax.broadcasted_iota(jnp.int32, (T, T), 0)
    c = lax.broadcasted_iota(jnp.int32, (T, T), 1)
    tril = (r >= c).astype(jnp.bfloat16)
    c1 = lax.dot_general(tril, oh1, (((1,), (0,)), ((), ())),
                         preferred_element_type=jnp.float32)
    c2 = lax.dot_general(tril, oh2, (((1,), (0,)), ((), ())),
                         preferred_element_type=jnp.float32)
    n1 = c1[T - 1:T, :]
    counts = n1 + c2[T - 1:T, :]
    lt = ((i1 < col).astype(jnp.bfloat16) + (i2 < col).astype(jnp.bfloat16))
    ones_row = jnp.ones((1, T), jnp.bfloat16)
    off = lax.dot_general(ones_row, lt, (((1,), (0,)), ((), ())),
                          preferred_element_type=jnp.float32)

    oh1f = oh1.astype(jnp.float32)
    oh2f = oh2.astype(jnp.float32)
    pos1 = jnp.sum(oh1f * (off + c1 - 1.0), axis=1, keepdims=True)
    pos2 = jnp.sum(oh2f * (off + n1 + c2 - 1.0), axis=1, keepdims=True)
    pos_ref[0:T, :] = pos1.astype(jnp.int32)
    pos_ref[T:A, :] = pos2.astype(jnp.int32)
    w_ref[0:T, :] = jnp.broadcast_to(m1 / ssum, (T, 128))
    w_ref[T:A, :] = jnp.broadcast_to(m2 / ssum, (T, 128))

    ecol1 = _fiota((1, EP), 1)
    starts = jnp.where(ecol1 == float(E), float(A), off)
    ends = jnp.where(ecol1 == float(E), float(NTOT), off + counts)
    onesT = jnp.ones((T, 1), jnp.bfloat16)
    offT = lax.dot_general(lt, onesT, (((0,), (0,)), ((), ())),
                           preferred_element_type=jnp.float32)
    countsT = lax.dot_general(oh1 + oh2, onesT, (((0,), (0,)), ((), ())),
                              preferred_element_type=jnp.float32)
    erowT = _fiota((EP, NB), 0)
    bcolT = _fiota((EP, NB), 1)
    startsT = jnp.where(erowT == float(E), float(A),
                        jnp.broadcast_to(offT, (EP, NB)))
    endsT = jnp.where(erowT == float(E), float(NTOT),
                      jnp.broadcast_to(offT + countsT, (EP, NB)))
    mmT = ((startsT < (bcolT + 1.0) * BLK) & (endsT > bcolT * BLK)
           & (endsT > startsT)).astype(jnp.bfloat16)
    cnt_row = lax.dot_general(jnp.ones((1, EP), jnp.bfloat16), mmT,
                              (((1,), (0,)), ((), ())),
                              preferred_element_type=jnp.float32)
    b1 = _fiota((NB, NB), 0)
    b2 = _fiota((NB, NB), 1)
    cbi_row = lax.dot_general(cnt_row.astype(jnp.bfloat16),
                              (b1 <= b2).astype(jnp.bfloat16),
                              (((1,), (0,)), ((), ())),
                              preferred_element_type=jnp.float32)
    cbe_row = cbi_row - cnt_row
    g_i = _fiota((G, 1), 0)
    gcol = _fiota((G, NB), 1)
    b_idx = jnp.sum((g_i >= cbi_row).astype(jnp.float32), axis=1,
                    keepdims=True)
    padm = b_idx >= float(NB)
    b_idx = jnp.where(padm, float(NB - 1), b_idx)
    ohb = (gcol == jnp.broadcast_to(b_idx, (G, NB))).astype(jnp.float32)
    j_g = g_i - jnp.sum(ohb * cbe_row, axis=1, keepdims=True)
    e1 = _fiota((EP, EP), 0)
    e2 = _fiota((EP, EP), 1)
    rexcT = lax.dot_general((e2 < e1).astype(jnp.bfloat16), mmT,
                            (((1,), (0,)), ((), ())),
                            preferred_element_type=jnp.float32)
    ohb_bf = ohb.astype(jnp.bfloat16)
    mg = lax.dot_general(ohb_bf, mmT, (((1,), (1,)), ((), ())),
                         preferred_element_type=jnp.float32)
    rg = lax.dot_general(ohb_bf, rexcT.astype(jnp.bfloat16),
                         (((1,), (1,)), ((), ())),
                         preferred_element_type=jnp.float32)
    ecolg = _fiota((G, EP), 1)
    esel = mg * (rg == jnp.broadcast_to(j_g, (G, EP))).astype(jnp.float32)
    e_idx = jnp.sum(esel * ecolg, axis=1, keepdims=True)
    e_idx = jnp.where(padm, float(EP - 1), e_idx)
    ohe = (ecolg == jnp.broadcast_to(e_idx, (G, EP))).astype(jnp.float32)
    st0 = jnp.sum(ohe * starts, axis=1, keepdims=True)
    en0 = jnp.sum(ohe * ends, axis=1, keepdims=True)
    st_g = jnp.where(padm, 0.0, jnp.maximum(st0, b_idx * BLK))
    en_g = jnp.where(padm, 0.0, jnp.minimum(en0, (b_idx + 1.0) * BLK))
    first = jnp.where(padm, 0.0, (j_g == 0.0).astype(jnp.float32))
    b_ref[...] = b_idx.astype(jnp.int32)
    e_ref[...] = e_idx.astype(jnp.int32)
    f_ref[...] = first.astype(jnp.int32)
    s_ref[...] = st_g.astype(jnp.int32)
    en_ref[...] = en_g.astype(jnp.int32)


def _run_router(probs64):
    return pl.pallas_call(
        _router_body,
        out_shape=[
            jax.ShapeDtypeStruct((A, 1), jnp.int32),
            jax.ShapeDtypeStruct((A, 128), jnp.float32),
            jax.ShapeDtypeStruct((G, 1), jnp.int32),
            jax.ShapeDtypeStruct((G, 1), jnp.int32),
            jax.ShapeDtypeStruct((G, 1), jnp.int32),
            jax.ShapeDtypeStruct((G, 1), jnp.int32),
            jax.ShapeDtypeStruct((G, 1), jnp.int32),
        ],
    )(probs64)


def _make_scatter_kernel():
    mesh = plsc.VectorSubcoreMesh(core_axis_name="c", subcore_axis_name="s")

    @functools.partial(
        pl.kernel, mesh=mesh,
        out_type=[
            jax.ShapeDtypeStruct((NTOT, D), jnp.float32),
            jax.ShapeDtypeStruct((NTOT, 128), jnp.float32),
        ],
        scratch_types=[
            pltpu.VMEM((48,), jnp.int32),
            pltpu.VMEM((48,), jnp.int32),
            pltpu.VMEM((32,), jnp.int32),
            pltpu.VMEM((48, 128), jnp.float32),
            pltpu.VMEM((48, 128), jnp.float32),
            pltpu.VMEM((32, 128), jnp.float32),
            pltpu.VMEM((48, D), jnp.float32),
            pltpu.VMEM((48, D), jnp.float32),
            pltpu.SemaphoreType.DMA,
            pltpu.SemaphoreType.DMA,
        ],
    )
    def scatter_kernel(x_hbm, pos_hbm, wb_hbm, xs_hbm, ws_hbm,
                       idx0, idx1, idx2, wb0, wb1, wb2, rows_a, rows_b,
                       semx, semw):
        wid = lax.axis_index("s") * _NC + lax.axis_index("c")
        a0 = wid * 128
        tb0 = lax.rem(a0, T)
        tb1 = lax.rem(a0 + 48, T)
        tb2 = lax.rem(a0 + 96, T)
        sh = wid * 64
        pltpu.sync_copy(pos_hbm.at[pl.ds(a0, 48)], idx0)
        pltpu.sync_copy(pos_hbm.at[pl.ds(a0 + 48, 48)], idx1)
        pltpu.sync_copy(pos_hbm.at[pl.ds(a0 + 96, 32)], idx2)
        pltpu.sync_copy(wb_hbm.at[pl.ds(a0, 48)], wb0)
        pltpu.sync_copy(wb_hbm.at[pl.ds(a0 + 48, 48)], wb1)
        pltpu.sync_copy(wb_hbm.at[pl.ds(a0 + 96, 32)], wb2)
        dw0 = pltpu.async_copy(wb0, ws_hbm.at[idx0], semw)
        dw1 = pltpu.async_copy(wb1, ws_hbm.at[idx1], semw)
        dw2 = pltpu.async_copy(wb2, ws_hbm.at[idx2], semw)
        pltpu.sync_copy(x_hbm.at[pl.ds(tb0, 48)], rows_a)
        dx0 = pltpu.async_copy(rows_a, xs_hbm.at[idx0], semx)
        pltpu.sync_copy(x_hbm.at[pl.ds(tb1, 48)], rows_b)
        dx1 = pltpu.async_copy(rows_b, xs_hbm.at[idx1], semx)
        dx0.wait()
        pltpu.sync_copy(x_hbm.at[pl.ds(tb2, 32)], rows_a.at[pl.ds(0, 32)])
        dx2 = pltpu.async_copy(rows_a.at[pl.ds(0, 32)], xs_hbm.at[idx2], semx)
        dx1.wait()
        pltpu.sync_copy(x_hbm.at[pl.ds(sh, 48)], rows_b)
        dsh0 = pltpu.async_copy(rows_b, xs_hbm.at[pl.ds(A + sh, 48)], semx)
        dx2.wait()
        pltpu.sync_copy(x_hbm.at[pl.ds(sh + 48, 16)],
                        rows_a.at[pl.ds(0, 16)])
        dsh1 = pltpu.async_copy(rows_a.at[pl.ds(0, 16)],
                                xs_hbm.at[pl.ds(A + sh + 48, 16)], semx)
        dw0.wait()

        def _ones48(i, carry):
            for v in range(8):
                wb0[i, pl.ds(v * 16, 16)] = jnp.full((16,), 1.0, jnp.float32)
            return carry
        lax.fori_loop(0, 48, _ones48, 0)
        dws0 = pltpu.async_copy(wb0, ws_hbm.at[pl.ds(A + sh, 48)], semw)
        dw2.wait()

        def _ones16(i, carry):
            for v in range(8):
                wb2[i, pl.ds(v * 16, 16)] = jnp.full((16,), 1.0, jnp.float32)
            return carry
        lax.fori_loop(0, 16, _ones16, 0)
        dws1 = pltpu.async_copy(wb2.at[pl.ds(0, 16)],
                                ws_hbm.at[pl.ds(A + sh + 48, 16)], semw)
        dw1.wait()
        dsh0.wait()
        dsh1.wait()
        dws0.wait()
        dws1.wait()

    return scatter_kernel


def _gmm_body(bs, es, fs, ss, ens, x_ref, w_ref, rg_ref, ru_ref, rd_ref,
              sg_ref, su_ref, sd_ref, y_ref):
    g = pl.program_id(0)

    @pl.when(fs[g] == 1)
    def _():
        y_ref[...] = jnp.zeros_like(y_ref)

    xb = x_ref[...]
    row = bs[g] * BLK + lax.broadcasted_iota(jnp.int32, (BLK, D), 0)
    mask = (row >= ss[g]) & (row < ens[g])
    wcol = w_ref[...][:, 0:1]

    def ffn(gw, uw, dw):
        h = lax.dot_general(xb, gw, (((1,), (1,)), ((), ())),
                            preferred_element_type=jnp.float32)
        u = lax.dot_general(xb, uw, (((1,), (1,)), ((), ())),
                            preferred_element_type=jnp.float32)
        act = (h * jax.nn.sigmoid(h)) * u
        return lax.dot_general(act, dw, (((1,), (1,)), ((), ())),
                               preferred_element_type=jnp.float32)

    nonempty = ens[g] > ss[g]

    @pl.when((es[g] >= E) & nonempty)
    def _():
        y = ffn(sg_ref[0], su_ref[0], sd_ref[0])
        y_ref[...] = y_ref[...] + jnp.where(mask, y * wcol, 0.0)

    @pl.when((es[g] < E) & nonempty)
    def _():
        y = ffn(rg_ref[0], ru_ref[0], rd_ref[0])
        y_ref[...] = y_ref[...] + jnp.where(mask, y * wcol, 0.0)


def _run_gmm(b_idx, e_idx, first, st_g, en_g, xs, ws, routed_gate, routed_up,
             routed_down, shared_gate, shared_up, shared_down):
    grid_spec = pltpu.PrefetchScalarGridSpec(
        num_scalar_prefetch=5,
        grid=(G,),
        in_specs=[
            pl.BlockSpec((BLK, D), lambda g, b, e, f, s, en: (b[g], 0)),
            pl.BlockSpec((BLK, 128), lambda g, b, e, f, s, en: (b[g], 0)),
            pl.BlockSpec((1, F, D),
                         lambda g, b, e, f, s, en: (jnp.minimum(e[g], E - 1),
                                                    0, 0)),
            pl.BlockSpec((1, F, D),
                         lambda g, b, e, f, s, en: (jnp.minimum(e[g], E - 1),
                                                    0, 0)),
            pl.BlockSpec((1, D, F),
                         lambda g, b, e, f, s, en: (jnp.minimum(e[g], E - 1),
                                                    0, 0)),
            pl.BlockSpec((1, F, D), lambda g, b, e, f, s, en: (0, 0, 0)),
            pl.BlockSpec((1, F, D), lambda g, b, e, f, s, en: (0, 0, 0)),
            pl.BlockSpec((1, D, F), lambda g, b, e, f, s, en: (0, 0, 0)),
        ],
        out_specs=pl.BlockSpec((BLK, D), lambda g, b, e, f, s, en: (b[g], 0)),
    )
    return pl.pallas_call(
        _gmm_body,
        grid_spec=grid_spec,
        out_shape=jax.ShapeDtypeStruct((NTOT, D), jnp.float32),
        compiler_params=pltpu.CompilerParams(
            dimension_semantics=("arbitrary",)),
    )(b_idx, e_idx, first, st_g, en_g, xs, ws,
      routed_gate, routed_up, routed_down, shared_gate, shared_up, shared_down)


def _make_combine_kernel():
    mesh = plsc.VectorSubcoreMesh(core_axis_name="c", subcore_axis_name="s")

    @functools.partial(
        pl.kernel, mesh=mesh,
        out_type=jax.ShapeDtypeStruct((T, D), jnp.float32),
        scratch_types=[
            pltpu.VMEM((16,), jnp.int32), pltpu.VMEM((16,), jnp.int32),
            pltpu.VMEM((16, D), jnp.float32), pltpu.VMEM((16, D), jnp.float32),
            pltpu.VMEM((16, D), jnp.float32),
            pltpu.VMEM((16,), jnp.int32), pltpu.VMEM((16,), jnp.int32),
            pltpu.VMEM((16, D), jnp.float32), pltpu.VMEM((16, D), jnp.float32),
            pltpu.VMEM((16, D), jnp.float32),
            pltpu.SemaphoreType.DMA, pltpu.SemaphoreType.DMA,
            pltpu.SemaphoreType.DMA, pltpu.SemaphoreType.DMA,
        ],
    )
    def combine_kernel(y_hbm, pos_hbm, out_hbm,
                       i0a, i1a, y0a, y1a, ysha, i0b, i1b, y0b, y1b, yshb,
                       semga, semgb, semoa, semob):
        wid = lax.axis_index("s") * _NC + lax.axis_index("c")
        slots = ((i0a, i1a, y0a, y1a, ysha, semga, semoa),
                 (i0b, i1b, y0b, y1b, yshb, semgb, semob))
        gdesc = [None, None]
        odesc = [None, None]

        def issue(c):
            i0, i1, y0, y1, ysh, semg, _ = slots[c % 2]
            t0 = wid * 64 + c * 16
            pltpu.sync_copy(pos_hbm.at[pl.ds(t0, 16)], i0)
            pltpu.sync_copy(pos_hbm.at[pl.ds(T + t0, 16)], i1)
            gdesc[c % 2] = (
                pltpu.async_copy(y_hbm.at[i0], y0, semg),
                pltpu.async_copy(y_hbm.at[i1], y1, semg),
                pltpu.async_copy(y_hbm.at[pl.ds(A + t0, 16)], ysh, semg))

        issue(0)
        for c in range(4):
            if c + 1 < 4:
                if odesc[(c + 1) % 2] is not None:
                    odesc[(c + 1) % 2].wait()
                issue(c + 1)
            _, _, y0, y1, ysh, _, semo = slots[c % 2]
            for d in gdesc[c % 2]:
                d.wait()

            def _row(r, carry):
                def _vec(v, carry2):
                    for u_ in range(4):
                        o = v * 64 + u_ * 16
                        y0[r, pl.ds(o, 16)] = (y0[r, pl.ds(o, 16)]
                                               + y1[r, pl.ds(o, 16)]
                                               + ysh[r, pl.ds(o, 16)])
                    return carry2
                return lax.fori_loop(0, 16, _vec, carry)
            lax.fori_loop(0, 16, _row, 0)
            t0 = wid * 64 + c * 16
            odesc[c % 2] = pltpu.async_copy(y0, out_hbm.at[pl.ds(t0, 16)],
                                            semo)
        odesc[0].wait()
        odesc[1].wait()

    return combine_kernel


def kernel(x, shared_gate, shared_up, shared_down, routed_gate, routed_up,
           routed_down, routing_w, routing_b, routing_bias):
    x2 = x.reshape(T, D)
    logits = x2 @ routing_w.T + routing_b + routing_bias
    probs = jax.nn.sigmoid(logits)
    probs64 = jnp.concatenate([probs, jnp.full((T, 1), -1.0, jnp.float32)], 1)

    posc, wflat, b_idx, e_idx, first, st_g, en_g = _run_router(probs64)
    posf = posc.reshape(-1)
    b_idx = b_idx.reshape(-1)
    e_idx = e_idx.reshape(-1)
    first = first.reshape(-1)
    st_g = st_g.reshape(-1)
    en_g = en_g.reshape(-1)

    xs, ws = _make_scatter_kernel()(x2, posf, wflat)
    y = _run_gmm(b_idx, e_idx, first, st_g, en_g, xs, ws,
                 routed_gate, routed_up, routed_down,
                 shared_gate, shared_up, shared_down)
    out = _make_combine_kernel()(y, posf)
    return out.reshape(1, T, D)


# speedup vs baseline: 1.0070x; 1.0070x over previous
import functools

import jax
import jax.numpy as jnp
from jax import lax
from jax.experimental import pallas as pl
from jax.experimental.pallas import tpu as pltpu
from jax.experimental.pallas import tpu_sc as plsc

T = 2048
D = 1024
F = 512
E = 63
EP = 64
A = 2 * T
NTOT = A + T
BLK = 256
NB = NTOT // BLK
G = NB + EP

_NC = 2
_NS = 16
_NW = _NC * _NS


def _fiota(shape, dim):
    return lax.broadcasted_iota(jnp.int32, shape, dim).astype(jnp.float32)


def _router_body(p_ref, pos_ref, w_ref, b_ref, e_ref, f_ref, s_ref,
                 en_ref):
    probs = p_ref[...]
    col = lax.broadcasted_iota(jnp.int32, (T, EP), 1)

    big = jnp.int32(1 << 20)
    m1 = jnp.max(probs, axis=1, keepdims=True)
    i1 = jnp.min(jnp.where(probs == m1, col, big), axis=1, keepdims=True)
    pm = jnp.where(col == i1, -1.0, probs)
    m2 = jnp.max(pm, axis=1, keepdims=True)
    i2 = jnp.min(jnp.where(pm == m2, col, big), axis=1, keepdims=True)
    ssum = m1 + m2

    oh1 = (col == i1).astype(jnp.bfloat16)
    oh2 = (col == i2).astype(jnp.bfloat16)
    r = lax.broadcasted_iota(jnp.int32, (T, T), 0)
    c = lax.broadcasted_iota(jnp.int32, (T, T), 1)
    tril = (r >= c).astype(jnp.bfloat16)
    c1 = lax.dot_general(tril, oh1, (((1,), (0,)), ((), ())),
                         preferred_element_type=jnp.float32)
    c2 = lax.dot_general(tril, oh2, (((1,), (0,)), ((), ())),
                         preferred_element_type=jnp.float32)
    n1 = c1[T - 1:T, :]
    counts = n1 + c2[T - 1:T, :]
    lt = ((i1 < col).astype(jnp.bfloat16) + (i2 < col).astype(jnp.bfloat16))
    ones_row = jnp.ones((1, T), jnp.bfloat16)
    off = lax.dot_general(ones_row, lt, (((1,), (0,)), ((), ())),
                          preferred_element_type=jnp.float32)

    oh1f = oh1.astype(jnp.float32)
    oh2f = oh2.astype(jnp.float32)
    pos1 = jnp.sum(oh1f * (off + c1 - 1.0), axis=1, keepdims=True)
    pos2 = jnp.sum(oh2f * (off + n1 + c2 - 1.0), axis=1, keepdims=True)
    pos_ref[0:T, :] = pos1.astype(jnp.int32)
    pos_ref[T:A, :] = pos2.astype(jnp.int32)
    w_ref[0:T, :] = jnp.broadcast_to(m1 / ssum, (T, 128))
    w_ref[T:A, :] = jnp.broadcast_to(m2 / ssum, (T, 128))

    ecol1 = _fiota((1, EP), 1)
    starts = jnp.where(ecol1 == float(E), float(A), off)
    ends = jnp.where(ecol1 == float(E), float(NTOT), off + counts)
    onesT = jnp.ones((T, 1), jnp.bfloat16)
    offT = lax.dot_general(lt, onesT, (((0,), (0,)), ((), ())),
                           preferred_element_type=jnp.float32)
    countsT = lax.dot_general(oh1 + oh2, onesT, (((0,), (0,)), ((), ())),
                              preferred_element_type=jnp.float32)
    erowT = _fiota((EP, NB), 0)
    bcolT = _fiota((EP, NB), 1)
    startsT = jnp.where(erowT == float(E), float(A),
                        jnp.broadcast_to(offT, (EP, NB)))
    endsT = jnp.where(erowT == float(E), float(NTOT),
                      jnp.broadcast_to(offT + countsT, (EP, NB)))
    mmT = ((startsT < (bcolT + 1.0) * BLK) & (endsT > bcolT * BLK)
           & (endsT > startsT)).astype(jnp.bfloat16)
    cnt_row = lax.dot_general(jnp.ones((1, EP), jnp.bfloat16), mmT,
                              (((1,), (0,)), ((), ())),
                              preferred_element_type=jnp.float32)
    b1 = _fiota((NB, NB), 0)
    b2 = _fiota((NB, NB), 1)
    cbi_row = lax.dot_general(cnt_row.astype(jnp.bfloat16),
                              (b1 <= b2).astype(jnp.bfloat16),
                              (((1,), (0,)), ((), ())),
                              preferred_element_type=jnp.float32)
    cbe_row = cbi_row - cnt_row
    g_i = _fiota((G, 1), 0)
    gcol = _fiota((G, NB), 1)
    b_idx = jnp.sum((g_i >= cbi_row).astype(jnp.float32), axis=1,
                    keepdims=True)
    padm = b_idx >= float(NB)
    b_idx = jnp.where(padm, float(NB - 1), b_idx)
    ohb = (gcol == jnp.broadcast_to(b_idx, (G, NB))).astype(jnp.float32)
    j_g = g_i - jnp.sum(ohb * cbe_row, axis=1, keepdims=True)
    e1 = _fiota((EP, EP), 0)
    e2 = _fiota((EP, EP), 1)
    rexcT = lax.dot_general((e2 < e1).astype(jnp.bfloat16), mmT,
                            (((1,), (0,)), ((), ())),
                            preferred_element_type=jnp.float32)
    ohb_bf = ohb.astype(jnp.bfloat16)
    mg = lax.dot_general(ohb_bf, mmT, (((1,), (1,)), ((), ())),
                         preferred_element_type=jnp.float32)
    rg = lax.dot_general(ohb_bf, rexcT.astype(jnp.bfloat16),
                         (((1,), (1,)), ((), ())),
                         preferred_element_type=jnp.float32)
    ecolg = _fiota((G, EP), 1)
    esel = mg * (rg == jnp.broadcast_to(j_g, (G, EP))).astype(jnp.float32)
    e_idx = jnp.sum(esel * ecolg, axis=1, keepdims=True)
    e_idx = jnp.where(padm, float(EP - 1), e_idx)
    ohe = (ecolg == jnp.broadcast_to(e_idx, (G, EP))).astype(jnp.float32)
    st0 = jnp.sum(ohe * starts, axis=1, keepdims=True)
    en0 = jnp.sum(ohe * ends, axis=1, keepdims=True)
    st_g = jnp.where(padm, 0.0, jnp.maximum(st0, b_idx * BLK))
    en_g = jnp.where(padm, 0.0, jnp.minimum(en0, (b_idx + 1.0) * BLK))
    first = jnp.where(padm, 0.0, (j_g == 0.0).astype(jnp.float32))
    b_ref[...] = b_idx.astype(jnp.int32)
    e_ref[...] = e_idx.astype(jnp.int32)
    f_ref[...] = first.astype(jnp.int32)
    s_ref[...] = st_g.astype(jnp.int32)
    en_ref[...] = en_g.astype(jnp.int32)


def _run_router(probs64):
    return pl.pallas_call(
        _router_body,
        out_shape=[
            jax.ShapeDtypeStruct((A, 1), jnp.int32),
            jax.ShapeDtypeStruct((A, 128), jnp.float32),
            jax.ShapeDtypeStruct((G, 1), jnp.int32),
            jax.ShapeDtypeStruct((G, 1), jnp.int32),
            jax.ShapeDtypeStruct((G, 1), jnp.int32),
            jax.ShapeDtypeStruct((G, 1), jnp.int32),
            jax.ShapeDtypeStruct((G, 1), jnp.int32),
        ],
    )(probs64)


def _make_scatter_kernel():
    mesh = plsc.VectorSubcoreMesh(core_axis_name="c", subcore_axis_name="s")

    @functools.partial(
        pl.kernel, mesh=mesh,
        out_type=[
            jax.ShapeDtypeStruct((NTOT, D), jnp.float32),
            jax.ShapeDtypeStruct((NTOT, 128), jnp.float32),
        ],
        scratch_types=[
            pltpu.VMEM((48,), jnp.int32),
            pltpu.VMEM((48,), jnp.int32),
            pltpu.VMEM((32,), jnp.int32),
            pltpu.VMEM((48, 128), jnp.float32),
            pltpu.VMEM((48, 128), jnp.float32),
            pltpu.VMEM((32, 128), jnp.float32),
            pltpu.VMEM((48, D), jnp.float32),
            pltpu.VMEM((48, D), jnp.float32),
            pltpu.SemaphoreType.DMA,
            pltpu.SemaphoreType.DMA,
        ],
    )
    def scatter_kernel(x_hbm, pos_hbm, wb_hbm, xs_hbm, ws_hbm,
                       idx0, idx1, idx2, wb0, wb1, wb2, rows_a, rows_b,
                       semx, semw):
        wid = lax.axis_index("s") * _NC + lax.axis_index("c")
        a0 = wid * 128
        tb0 = lax.rem(a0, T)
        tb1 = lax.rem(a0 + 48, T)
        tb2 = lax.rem(a0 + 96, T)
        sh = wid * 64
        pltpu.sync_copy(pos_hbm.at[pl.ds(a0, 48)], idx0)
        pltpu.sync_copy(pos_hbm.at[pl.ds(a0 + 48, 48)], idx1)
        pltpu.sync_copy(pos_hbm.at[pl.ds(a0 + 96, 32)], idx2)
        pltpu.sync_copy(wb_hbm.at[pl.ds(a0, 48)], wb0)
        pltpu.sync_copy(wb_hbm.at[pl.ds(a0 + 48, 48)], wb1)
        pltpu.sync_copy(wb_hbm.at[pl.ds(a0 + 96, 32)], wb2)
        dw0 = pltpu.async_copy(wb0, ws_hbm.at[idx0], semw)
        dw1 = pltpu.async_copy(wb1, ws_hbm.at[idx1], semw)
        dw2 = pltpu.async_copy(wb2, ws_hbm.at[idx2], semw)
        pltpu.sync_copy(x_hbm.at[pl.ds(tb0, 48)], rows_a)
        dx0 = pltpu.async_copy(rows_a, xs_hbm.at[idx0], semx)
        pltpu.sync_copy(x_hbm.at[pl.ds(tb1, 48)], rows_b)
        dx1 = pltpu.async_copy(rows_b, xs_hbm.at[idx1], semx)
        dx0.wait()
        pltpu.sync_copy(x_hbm.at[pl.ds(tb2, 32)], rows_a.at[pl.ds(0, 32)])
        dx2 = pltpu.async_copy(rows_a.at[pl.ds(0, 32)], xs_hbm.at[idx2], semx)
        dx1.wait()
        pltpu.sync_copy(x_hbm.at[pl.ds(sh, 48)], rows_b)
        dsh0 = pltpu.async_copy(rows_b, xs_hbm.at[pl.ds(A + sh, 48)], semx)
        dx2.wait()
        pltpu.sync_copy(x_hbm.at[pl.ds(sh + 48, 16)],
                        rows_a.at[pl.ds(0, 16)])
        dsh1 = pltpu.async_copy(rows_a.at[pl.ds(0, 16)],
                                xs_hbm.at[pl.ds(A + sh + 48, 16)], semx)
        dw0.wait()

        def _ones48(i, carry):
            for v in range(8):
                wb0[i, pl.ds(v * 16, 16)] = jnp.full((16,), 1.0, jnp.float32)
            return carry
        lax.fori_loop(0, 48, _ones48, 0)
        dws0 = pltpu.async_copy(wb0, ws_hbm.at[pl.ds(A + sh, 48)], semw)
        dw2.wait()

        def _ones16(i, carry):
            for v in range(8):
                wb2[i, pl.ds(v * 16, 16)] = jnp.full((16,), 1.0, jnp.float32)
            return carry
        lax.fori_loop(0, 16, _ones16, 0)
        dws1 = pltpu.async_copy(wb2.at[pl.ds(0, 16)],
                                ws_hbm.at[pl.ds(A + sh + 48, 16)], semw)
        dw1.wait()
        dsh0.wait()
        dsh1.wait()
        dws0.wait()
        dws1.wait()

    return scatter_kernel


def _gmm_body(bs, es, fs, ss, ens, x_ref, w_ref, rg_ref, ru_ref, rd_ref,
              sg_ref, su_ref, sd_ref, y_ref):
    g = pl.program_id(0)

    @pl.when(fs[g] == 1)
    def _():
        y_ref[...] = jnp.zeros_like(y_ref)

    xb = x_ref[...]
    row = bs[g] * BLK + lax.broadcasted_iota(jnp.int32, (BLK, D), 0)
    mask = (row >= ss[g]) & (row < ens[g])
    wcol = w_ref[...][:, 0:1]

    def ffn(gw, uw, dw):
        h = lax.dot_general(xb, gw, (((1,), (1,)), ((), ())),
                            preferred_element_type=jnp.float32)
        u = lax.dot_general(xb, uw, (((1,), (1,)), ((), ())),
                            preferred_element_type=jnp.float32)
        act = (h * jax.nn.sigmoid(h)) * u
        return lax.dot_general(act, dw, (((1,), (1,)), ((), ())),
                               preferred_element_type=jnp.float32)

    nonempty = ens[g] > ss[g]

    @pl.when((es[g] >= E) & nonempty)
    def _():
        y = ffn(sg_ref[0], su_ref[0], sd_ref[0])
        y_ref[...] = y_ref[...] + jnp.where(mask, y * wcol, 0.0)

    @pl.when((es[g] < E) & nonempty)
    def _():
        y = ffn(rg_ref[0], ru_ref[0], rd_ref[0])
        y_ref[...] = y_ref[...] + jnp.where(mask, y * wcol, 0.0)


def _run_gmm(b_idx, e_idx, first, st_g, en_g, xs, ws, routed_gate, routed_up,
             routed_down, shared_gate, shared_up, shared_down):
    grid_spec = pltpu.PrefetchScalarGridSpec(
        num_scalar_prefetch=5,
        grid=(G,),
        in_specs=[
            pl.BlockSpec((BLK, D), lambda g, b, e, f, s, en: (b[g], 0)),
            pl.BlockSpec((BLK, 128), lambda g, b, e, f, s, en: (b[g], 0)),
            pl.BlockSpec((1, F, D),
                         lambda g, b, e, f, s, en: (jnp.minimum(e[g], E - 1),
                                                    0, 0)),
            pl.BlockSpec((1, F, D),
                         lambda g, b, e, f, s, en: (jnp.minimum(e[g], E - 1),
                                                    0, 0)),
            pl.BlockSpec((1, D, F),
                         lambda g, b, e, f, s, en: (jnp.minimum(e[g], E - 1),
                                                    0, 0)),
            pl.BlockSpec((1, F, D), lambda g, b, e, f, s, en: (0, 0, 0)),
            pl.BlockSpec((1, F, D), lambda g, b, e, f, s, en: (0, 0, 0)),
            pl.BlockSpec((1, D, F), lambda g, b, e, f, s, en: (0, 0, 0)),
        ],
        out_specs=pl.BlockSpec((BLK, D), lambda g, b, e, f, s, en: (b[g], 0)),
    )
    return pl.pallas_call(
        _gmm_body,
        grid_spec=grid_spec,
        out_shape=jax.ShapeDtypeStruct((NTOT, D), jnp.float32),
        compiler_params=pltpu.CompilerParams(
            dimension_semantics=("arbitrary",)),
    )(b_idx, e_idx, first, st_g, en_g, xs, ws,
      routed_gate, routed_up, routed_down, shared_gate, shared_up, shared_down)


def _make_combine_kernel():
    mesh = plsc.VectorSubcoreMesh(core_axis_name="c", subcore_axis_name="s")

    @functools.partial(
        pl.kernel, mesh=mesh,
        out_type=jax.ShapeDtypeStruct((T, D), jnp.float32),
        scratch_types=[
            pltpu.VMEM((16,), jnp.int32), pltpu.VMEM((16,), jnp.int32),
            pltpu.VMEM((16, D), jnp.float32), pltpu.VMEM((16, D), jnp.float32),
            pltpu.VMEM((16, D), jnp.float32),
            pltpu.VMEM((16,), jnp.int32), pltpu.VMEM((16,), jnp.int32),
            pltpu.VMEM((16, D), jnp.float32), pltpu.VMEM((16, D), jnp.float32),
            pltpu.VMEM((16, D), jnp.float32),
            pltpu.SemaphoreType.DMA, pltpu.SemaphoreType.DMA,
            pltpu.SemaphoreType.DMA, pltpu.SemaphoreType.DMA,
        ],
    )
    def combine_kernel(y_hbm, pos_hbm, out_hbm,
                       i0a, i1a, y0a, y1a, ysha, i0b, i1b, y0b, y1b, yshb,
                       semga, semgb, semoa, semob):
        wid = lax.axis_index("s") * _NC + lax.axis_index("c")
        slots = ((i0a, i1a, y0a, y1a, ysha, semga, semoa),
                 (i0b, i1b, y0b, y1b, yshb, semgb, semob))
        gdesc = [None, None]
        odesc = [None, None]

        def issue(c):
            i0, i1, y0, y1, ysh, semg, _ = slots[c % 2]
            t0 = wid * 64 + c * 16
            pltpu.sync_copy(pos_hbm.at[pl.ds(t0, 16)], i0)
            pltpu.sync_copy(pos_hbm.at[pl.ds(T + t0, 16)], i1)
            gdesc[c % 2] = (
                pltpu.async_copy(y_hbm.at[i0], y0, semg),
                pltpu.async_copy(y_hbm.at[i1], y1, semg),
                pltpu.async_copy(y_hbm.at[pl.ds(A + t0, 16)], ysh, semg))

        issue(0)
        for c in range(4):
            if c + 1 < 4:
                if odesc[(c + 1) % 2] is not None:
                    odesc[(c + 1) % 2].wait()
                issue(c + 1)
            _, _, y0, y1, ysh, _, semo = slots[c % 2]
            for d in gdesc[c % 2]:
                d.wait()

            def _row(r, carry):
                def _vec(v, carry2):
                    for u_ in range(4):
                        o = v * 64 + u_ * 16
                        y0[r, pl.ds(o, 16)] = (y0[r, pl.ds(o, 16)]
                                               + y1[r, pl.ds(o, 16)]
                                               + ysh[r, pl.ds(o, 16)])
                    return carry2
                return lax.fori_loop(0, 16, _vec, carry)
            lax.fori_loop(0, 16, _row, 0)
            t0 = wid * 64 + c * 16
            odesc[c % 2] = pltpu.async_copy(y0, out_hbm.at[pl.ds(t0, 16)],
                                            semo)
        odesc[0].wait()
        odesc[1].wait()

    return combine_kernel


def kernel(x, shared_gate, shared_up, shared_down, routed_gate, routed_up,
           routed_down, routing_w, routing_b, routing_bias):
    x2 = x.reshape(T, D)
    logits = x2 @ routing_w.T + routing_b + routing_bias
    probs = jax.nn.sigmoid(logits)
    probs64 = jnp.concatenate([probs, jnp.full((T, 1), -1.0, jnp.float32)], 1)

    posc, wflat, b_idx, e_idx, first, st_g, en_g = _run_router(probs64)
    posf = posc.reshape(-1)
    b_idx = b_idx.reshape(-1)
    e_idx = e_idx.reshape(-1)
    first = first.reshape(-1)
    st_g = st_g.reshape(-1)
    en_g = en_g.reshape(-1)

    xs, ws = _make_scatter_kernel()(x2, posf, wflat)
    y = _run_gmm(b_idx, e_idx, first, st_g, en_g, xs, ws,
                 routed_gate, routed_up, routed_down,
                 shared_gate, shared_up, shared_down)
    out = _make_combine_kernel()(y, posf)
    return out.reshape(1, T, D)
